# baseline (speedup 1.0000x reference)
import jax
import jax.numpy as jnp
from jax import lax
from jax.experimental import pallas as pl
from jax.experimental.pallas import tpu as pltpu
from jax.experimental.pallas import tpu_sc as plsc

SEQ = 16384
Q = 32
D = 4 * Q
NC = 2
NS = 16
NW = NC * NS
B_PER_W = SEQ // NW
L = 16
NGRP = B_PER_W // L
NPH = 2
GPP = NGRP // NPH
RPP = B_PER_W // NPH
TAB_ROWS = (24, 7, 12, 10)


def _body(h, d, m, y, wh, wd, wm, wy, out,
          th_v, td_v, tm_v, ty_v, ih_v, id_v, im_v, iy_v, rows_v,
          tsem, isem, wsem):
    wid = lax.axis_index("s") * NC + lax.axis_index("c")
    base = wid * B_PER_W
    tabs_h = (wh, wd, wm, wy)
    tabs_v = (th_v, td_v, tm_v, ty_v)
    idx_h = (h, d, m, y)
    idx_v = (ih_v, id_v, im_v, iy_v)
    copies = [pltpu.async_copy(tabs_h[j], tabs_v[j], tsem) for j in range(4)]
    copies += [pltpu.async_copy(idx_h[j].at[wid], idx_v[j], isem) for j in range(4)]
    for c in copies:
        c.wait()

    writes = []
    for p in range(NPH):
        @plsc.parallel_loop(p * GPP, (p + 1) * GPP, unroll=4)
        def grp(g):
            ivecs = [idx_v[j][pl.ds(g * L, L)] * Q for j in range(4)]
            for k in range(L):
                srcs = [ivecs[j][k] for j in range(4)]
                vals = [tabs_v[j][pl.ds(srcs[j] + c2 * L, L)]
                        for j in range(4) for c2 in range(Q // L)]
                dst = (g * L + k) * D
                for u in range(2 * 4):
                    rows_v[pl.ds(dst + u * L, L)] = vals[u]

        writes.append(pltpu.async_copy(
            rows_v.at[pl.ds(p * RPP * D, RPP * D)],
            out.at[pl.ds((base + p * RPP) * D, RPP * D)], wsem))
    for w in writes:
        w.wait()


_sc_call = pl.kernel(
    _body,
    out_type=jax.ShapeDtypeStruct((SEQ * D,), jnp.float32),
    mesh=plsc.VectorSubcoreMesh(core_axis_name="c", subcore_axis_name="s"),
    scratch_types=[
        pltpu.VMEM((TAB_ROWS[0] * Q,), jnp.float32),
        pltpu.VMEM((TAB_ROWS[1] * Q,), jnp.float32),
        pltpu.VMEM((TAB_ROWS[2] * Q,), jnp.float32),
        pltpu.VMEM((TAB_ROWS[3] * Q,), jnp.float32),
        pltpu.VMEM((B_PER_W,), jnp.int32),
        pltpu.VMEM((B_PER_W,), jnp.int32),
        pltpu.VMEM((B_PER_W,), jnp.int32),
        pltpu.VMEM((B_PER_W,), jnp.int32),
        pltpu.VMEM((B_PER_W * D,), jnp.float32),
        pltpu.SemaphoreType.DMA,
        pltpu.SemaphoreType.DMA,
        pltpu.SemaphoreType.DMA,
    ],
    compiler_params=pltpu.CompilerParams(
        use_tc_tiling_on_sc=False, needs_layout_passes=False),
)


def kernel(hours, days, months, years, W_hour, W_day, W_month, W_year):
    h = hours.astype(jnp.int32).reshape(NW, B_PER_W)
    d = days.astype(jnp.int32).reshape(NW, B_PER_W)
    m = months.astype(jnp.int32).reshape(NW, B_PER_W)
    y = years.astype(jnp.int32).reshape(NW, B_PER_W)
    out = _sc_call(h, d, m, y,
                   W_hour.reshape(-1), W_day.reshape(-1),
                   W_month.reshape(-1), W_year.reshape(-1))
    return out.reshape(SEQ, D)


# speedup vs baseline: 1.0079x; 1.0079x over previous
import jax
import jax.numpy as jnp
from jax import lax
from jax.experimental import pallas as pl
from jax.experimental.pallas import tpu as pltpu
from jax.experimental.pallas import tpu_sc as plsc

SEQ = 16384
Q = 32
D = 4 * Q
NC = 2
NS = 16
NW = NC * NS
B_PER_W = SEQ // NW
L = 16
NGRP = B_PER_W // L
NPH = 2
GPP = NGRP // NPH
RPP = B_PER_W // NPH
TAB_ROWS = (24, 7, 12, 10)


def _body(h, d, m, y, wh, wd, wm, wy, out,
          th_v, td_v, tm_v, ty_v, ih_v, id_v, im_v, iy_v, rows_v,
          tsem, isem, wsem):
    wid = lax.axis_index("s") * NC + lax.axis_index("c")
    base = wid * B_PER_W
    tabs_h = (wh, wd, wm, wy)
    tabs_v = (th_v, td_v, tm_v, ty_v)
    idx_h = (h, d, m, y)
    idx_v = (ih_v, id_v, im_v, iy_v)
    copies = [pltpu.async_copy(tabs_h[j], tabs_v[j], tsem) for j in range(4)]
    copies += [pltpu.async_copy(idx_h[j].at[wid], idx_v[j], isem) for j in range(4)]
    for c in copies:
        c.wait()

    writes = []
    for p in range(NPH):
        @plsc.parallel_loop(p * GPP, (p + 1) * GPP, unroll=2)
        def grp(g):
            ivecs = [idx_v[j][pl.ds(g * L, L)] * Q for j in range(4)]
            for k in range(L):
                srcs = [ivecs[j][k] for j in range(4)]
                vals = [tabs_v[j][pl.ds(srcs[j] + c2 * L, L)]
                        for j in range(4) for c2 in range(Q // L)]
                dst = (g * L + k) * D
                for u in range(2 * 4):
                    rows_v[pl.ds(dst + u * L, L)] = vals[u]

        writes.append(pltpu.async_copy(
            rows_v.at[pl.ds(p * RPP * D, RPP * D)],
            out.at[pl.ds((base + p * RPP) * D, RPP * D)], wsem))
    for w in writes:
        w.wait()


_sc_call = pl.kernel(
    _body,
    out_type=jax.ShapeDtypeStruct((SEQ * D,), jnp.float32),
    mesh=plsc.VectorSubcoreMesh(core_axis_name="c", subcore_axis_name="s"),
    scratch_types=[
        pltpu.VMEM((TAB_ROWS[0] * Q,), jnp.float32),
        pltpu.VMEM((TAB_ROWS[1] * Q,), jnp.float32),
        pltpu.VMEM((TAB_ROWS[2] * Q,), jnp.float32),
        pltpu.VMEM((TAB_ROWS[3] * Q,), jnp.float32),
        pltpu.VMEM((B_PER_W,), jnp.int32),
        pltpu.VMEM((B_PER_W,), jnp.int32),
        pltpu.VMEM((B_PER_W,), jnp.int32),
        pltpu.VMEM((B_PER_W,), jnp.int32),
        pltpu.VMEM((B_PER_W * D,), jnp.float32),
        pltpu.SemaphoreType.DMA,
        pltpu.SemaphoreType.DMA,
        pltpu.SemaphoreType.DMA,
    ],
    compiler_params=pltpu.CompilerParams(
        use_tc_tiling_on_sc=False, needs_layout_passes=False),
)


def kernel(hours, days, months, years, W_hour, W_day, W_month, W_year):
    h = hours.astype(jnp.int32).reshape(NW, B_PER_W)
    d = days.astype(jnp.int32).reshape(NW, B_PER_W)
    m = months.astype(jnp.int32).reshape(NW, B_PER_W)
    y = years.astype(jnp.int32).reshape(NW, B_PER_W)
    out = _sc_call(h, d, m, y,
                   W_hour.reshape(-1), W_day.reshape(-1),
                   W_month.reshape(-1), W_year.reshape(-1))
    return out.reshape(SEQ, D)
